# baseline (speedup 1.0000x reference)
import functools

import jax
import jax.numpy as jnp
from jax import lax
from jax.experimental import pallas as pl
from jax.experimental.pallas import tpu as pltpu
from jax.experimental.pallas import tpu_sc as plsc

NUSER = 1000000
NITEM = 100000
HID = 64
BATCH = 4096

_info = plsc.get_sparse_core_info()
_NC, _NS = _info.num_cores, _info.num_subcores
_NW = _NC * _NS
_BPW = BATCH // _NW


@functools.partial(
    pl.kernel,
    mesh=plsc.VectorSubcoreMesh(core_axis_name="c", subcore_axis_name="s"),
    out_type=[
        jax.ShapeDtypeStruct((BATCH, HID), jnp.float32),
        jax.ShapeDtypeStruct((BATCH, HID), jnp.float32),
    ],
    scratch_types=[
        pltpu.VMEM((_BPW,), jnp.int32),
        pltpu.VMEM((_BPW, HID), jnp.float32),
        pltpu.VMEM((_BPW,), jnp.int32),
        pltpu.VMEM((_BPW, HID), jnp.float32),
        pltpu.SemaphoreType.DMA,
        pltpu.SemaphoreType.DMA,
    ],
    compiler_params=pltpu.CompilerParams(use_tc_tiling_on_sc=False),
)
def _lookup(user_hbm, item_hbm, uf_hbm, if_hbm, uout_hbm, iout_hbm,
            uidx_v, urows_v, iidx_v, irows_v, usem, isem):
    wid = lax.axis_index("s") * _NC + lax.axis_index("c")
    base = wid * _BPW
    pltpu.sync_copy(user_hbm.at[pl.ds(base, _BPW)], uidx_v)
    pltpu.sync_copy(item_hbm.at[pl.ds(base, _BPW)], iidx_v)
    ucopy = pltpu.async_copy(uf_hbm.at[uidx_v], urows_v, usem)
    icopy = pltpu.async_copy(if_hbm.at[iidx_v], irows_v, isem)
    ucopy.wait()
    uw = pltpu.async_copy(urows_v, uout_hbm.at[pl.ds(base, _BPW)], usem)
    icopy.wait()
    iw = pltpu.async_copy(irows_v, iout_hbm.at[pl.ds(base, _BPW)], isem)
    uw.wait()
    iw.wait()


def kernel(user, item, user_factor, item_factor):
    user = user.astype(jnp.int32)
    item = item.astype(jnp.int32)
    user_emb, item_emb = _lookup(user, item, user_factor, item_factor)
    return (user_emb, item_emb)
